# Initial kernel scaffold; baseline (speedup 1.0000x reference)
import jax
import jax.numpy as jnp
from jax.experimental import pallas as pl


def kernel(scores, bbox_deltas, im_info):
    raise NotImplementedError("write your pallas kernel here")


# speedup vs baseline: 108.9783x; 108.9783x over previous
import numpy as np
import jax
import jax.numpy as jnp
from jax import lax
from jax.experimental import pallas as pl
from jax.experimental.pallas import tpu as pltpu

_FPN_ANCHOR_SCALES = [32, 64, 128, 256, 512]
_FPN_FEAT_STRIDES = [4, 8, 16, 32, 64]
_ANCHOR_RATIOS = [0.5, 1.0, 2.0]
_IM_SIZE = 512
_PRE = 6000
_POST = 300
_TH = 0.7
_LANES = 128
_NB = 48
_PAD_N = _NB * _LANES
_HIGH = lax.Precision.HIGHEST


def _gen_anchors() -> np.ndarray:
    all_boxes = []
    ratios = np.array(_ANCHOR_RATIOS, dtype=np.float64)
    for scale, stride in zip(_FPN_ANCHOR_SCALES, _FPN_FEAT_STRIDES):
        fh = _IM_SIZE // stride
        fw = _IM_SIZE // stride
        heights = scale / np.sqrt(ratios)
        widths = scale * np.sqrt(ratios)
        shifts_y = np.arange(0, fh) * stride
        shifts_x = np.arange(0, fw) * stride
        sx, sy = np.meshgrid(shifts_x, shifts_y)
        box_w, box_cx = np.meshgrid(widths, sx.flatten())
        box_h, box_cy = np.meshgrid(heights, sy.flatten())
        boxes = np.stack([box_cx - 0.5 * box_w, box_cy - 0.5 * box_h,
                          box_cx + 0.5 * box_w, box_cy + 0.5 * box_h],
                         axis=2).reshape(-1, 4)
        all_boxes.append(boxes)
    return np.concatenate(all_boxes, axis=0).astype(np.float32)


_ANCHORS = _gen_anchors()


def _tr(x):
    eye = jnp.eye(x.shape[0], dtype=jnp.float32)
    return lax.dot_general(x, eye, (((0,), (0,)), ((), ())), precision=_HIGH)


def _nms_body(im_ref, anch_ref, dl_ref, dr_ref, out_l_ref, out_r_ref,
              bs_l, bs_r, kp_l, kp_r, acc_l, acc_r, cnt):
    i = pl.program_id(0)
    cnt[0] = 0
    acc_l[...] = jnp.zeros(acc_l.shape, jnp.float32)
    acc_r[...] = jnp.zeros(acc_r.shape, jnp.float32)
    imx = im_ref[0, 0:1, :]
    imy = im_ref[0, 1:2, :]

    iota_r = lax.broadcasted_iota(jnp.int32, (_LANES, _LANES), 0)
    iota_c = lax.broadcasted_iota(jnp.int32, (_LANES, _LANES), 1)
    lt_strict = jnp.where(iota_r > iota_c, 1.0, 0.0).astype(jnp.float32)
    sub_iota = lax.broadcasted_iota(jnp.int32, (_LANES, 1), 0)
    q_iota = lax.broadcasted_iota(
        jnp.int32, (3 * _LANES, _LANES), 0).astype(jnp.float32)

    def decode_side(d_ref, bs_ref, k):
        a = anch_ref[0, k]
        d = d_ref[0, k]
        x1a, y1a, x2a, y2a = a[1:2], a[2:3], a[3:4], a[4:5]
        dx, dy, dw, dh = d[1:2], d[2:3], d[3:4], d[4:5]
        w = x2a - x1a + 1.0
        h = y2a - y1a + 1.0
        cx = x1a + 0.5 * w
        cy = y1a + 0.5 * h
        pcx = dx * w + cx
        pcy = dy * h + cy
        pw = jnp.exp(dw) * w
        ph = jnp.exp(dh) * h
        px1 = jnp.clip(pcx - 0.5 * pw, 0.0, imx)
        py1 = jnp.clip(pcy - 0.5 * ph, 0.0, imy)
        px2 = jnp.clip(pcx + 0.5 * pw, 0.0, imx)
        py2 = jnp.clip(pcy + 0.5 * ph, 0.0, imy)
        rows = jnp.concatenate(
            [jnp.zeros((1, _LANES), jnp.float32), px1, py1, px2, py2,
             jnp.zeros((3, _LANES), jnp.float32)], axis=0)
        bs_ref[pl.ds(k, 1)] = rows.reshape(1, 8, _LANES)
        return rows

    def side_keep(rows, bs_ref, kp_ref, k, avail0):
        bT = _tr(rows)
        x1c, y1c = bT[:, 1:2], bT[:, 2:3]
        x2c, y2c = bT[:, 3:4], bT[:, 4:5]
        area_c = (x2c - x1c) * (y2c - y1c)

        def iou_vs_rows(br):
            x1r, y1r, x2r, y2r = br[1:2], br[2:3], br[3:4], br[4:5]
            area_r = (x2r - x1r) * (y2r - y1r)
            xx1 = jnp.maximum(x1c, x1r)
            yy1 = jnp.maximum(y1c, y1r)
            xx2 = jnp.minimum(x2c, x2r)
            yy2 = jnp.minimum(y2c, y2r)
            iw = jnp.maximum(xx2 - xx1, 0.0)
            ih = jnp.maximum(yy2 - yy1, 0.0)
            inter = iw * ih
            union = area_c + area_r - inter
            return inter / jnp.maximum(union, 1e-9)

        def jstep(j, ext):
            br = bs_ref[0 + j]
            iou = iou_vs_rows(br)
            krow = kp_ref[pl.ds(j, 1), :]
            supp = jnp.where((iou > _TH) & (krow > 0.0), 1.0, 0.0)
            return jnp.maximum(ext, jnp.max(supp, axis=1, keepdims=True))

        ext = lax.fori_loop(0, k, jstep, jnp.zeros((_LANES, 1), jnp.float32))
        avail = jnp.where((avail0 > 0.0) & (ext == 0.0), 1.0, 0.0)

        iou_cc = iou_vs_rows(rows)
        mlow = jnp.where(iou_cc > _TH, 1.0, 0.0) * lt_strict

        def fcond(c):
            return jnp.logical_not(c[1])

        def fbody(c):
            s, _ = c
            tmp = avail * (1.0 - s)
            s2 = jnp.where(
                lax.dot_general(mlow, tmp, (((1,), (0,)), ((), ())),
                                precision=_HIGH) > 0.0, 1.0, 0.0)
            return (s2, jnp.all(s2 == s))

        s0 = jnp.zeros((_LANES, 1), jnp.float32)
        s_fin, _ = lax.while_loop(fcond, fbody, (s0, jnp.asarray(False)))
        keep = avail * (1.0 - s_fin)
        return keep, bT

    def block_step(k, carry):
        @pl.when(cnt[0] < _POST)
        def _():
            avail0 = jnp.where(sub_iota + _LANES * k < _PRE, 1.0, 0.0)
            rows_l = decode_side(dl_ref, bs_l, k)
            rows_r = decode_side(dr_ref, bs_r, k)
            keep_l, bT_l = side_keep(rows_l, bs_l, kp_l, k, avail0)
            keep_r, bT_r = side_keep(rows_r, bs_r, kp_r, k, avail0)
            joint = keep_l * keep_r
            pos = lax.dot_general(lt_strict, joint, (((1,), (0,)), ((), ())),
                                  precision=_HIGH) + cnt[0].astype(jnp.float32)
            x = jnp.concatenate(
                [keep_l, keep_r, joint, pos,
                 jnp.zeros((_LANES, 4), jnp.float32)], axis=1)
            r = _tr(x)
            kp_l[pl.ds(k, 1), :] = r[0:1]
            kp_r[pl.ds(k, 1), :] = r[1:2]
            jrow = r[2:3]
            prow = r[3:4]
            onehot = jnp.where((q_iota == prow) & (jrow > 0.0), 1.0, 0.0)
            acc_l[...] += lax.dot_general(
                onehot, bT_l, (((1,), (0,)), ((), ())), precision=_HIGH)
            acc_r[...] += lax.dot_general(
                onehot, bT_r, (((1,), (0,)), ((), ())), precision=_HIGH)
            cnt[0] = cnt[0] + jnp.sum(joint).astype(jnp.int32)
        return carry

    lax.fori_loop(0, _NB, block_step, 0)

    lane5 = lax.broadcasted_iota(jnp.int32, (_POST, 8), 1)
    bi = i.astype(jnp.float32)
    final_l = jnp.where(lane5 == 0, bi, acc_l[0:_POST, :])
    final_r = jnp.where(lane5 == 0, bi, acc_r[0:_POST, :])
    out_l_ref[0] = final_l[:, 0:5]
    out_r_ref[0] = final_r[:, 0:5]


def _run(scores, bbox_deltas, im_info):
    B = scores.shape[0]
    sf = scores[:, :, 1]
    dl = bbox_deltas[..., :4]
    dr = jnp.stack([bbox_deltas[..., 4], bbox_deltas[..., 1],
                    bbox_deltas[..., 5], bbox_deltas[..., 3]], axis=-1)
    order = jnp.argsort(-sf, axis=1)[:, :_PRE]
    anch = jnp.broadcast_to(jnp.asarray(_ANCHORS)[None], (B,) + _ANCHORS.shape)
    anch_g = jnp.take_along_axis(anch, order[..., None], axis=1)
    dl_g = jnp.take_along_axis(dl, order[..., None], axis=1)
    dr_g = jnp.take_along_axis(dr, order[..., None], axis=1)

    def to_blocks(x):
        x = jnp.pad(x, ((0, 0), (0, _PAD_N - _PRE), (0, 0)))
        x = x.transpose(0, 2, 1).reshape(B, 4, _NB, _LANES).transpose(0, 2, 1, 3)
        return jnp.pad(x, ((0, 0), (0, 0), (1, 3), (0, 0)))

    imax = jnp.stack([im_info[:, 1] - 1.0, im_info[:, 0] - 1.0], axis=1)
    imax_b = jnp.broadcast_to(
        jnp.pad(imax, ((0, 0), (0, 6)))[:, :, None], (B, 8, _LANES))

    out_l, out_r = pl.pallas_call(
        _nms_body,
        grid=(B,),
        in_specs=[
            pl.BlockSpec((1, 8, _LANES), lambda i: (i, 0, 0)),
            pl.BlockSpec((1, _NB, 8, _LANES), lambda i: (i, 0, 0, 0)),
            pl.BlockSpec((1, _NB, 8, _LANES), lambda i: (i, 0, 0, 0)),
            pl.BlockSpec((1, _NB, 8, _LANES), lambda i: (i, 0, 0, 0)),
        ],
        out_specs=[
            pl.BlockSpec((1, _POST, 5), lambda i: (i, 0, 0)),
            pl.BlockSpec((1, _POST, 5), lambda i: (i, 0, 0)),
        ],
        out_shape=[
            jax.ShapeDtypeStruct((B, _POST, 5), jnp.float32),
            jax.ShapeDtypeStruct((B, _POST, 5), jnp.float32),
        ],
        scratch_shapes=[
            pltpu.VMEM((_NB, 8, _LANES), jnp.float32),
            pltpu.VMEM((_NB, 8, _LANES), jnp.float32),
            pltpu.VMEM((_NB, _LANES), jnp.float32),
            pltpu.VMEM((_NB, _LANES), jnp.float32),
            pltpu.VMEM((3 * _LANES, 8), jnp.float32),
            pltpu.VMEM((3 * _LANES, 8), jnp.float32),
            pltpu.SMEM((1,), jnp.int32),
        ],
    )(imax_b, to_blocks(anch_g), to_blocks(dl_g), to_blocks(dr_g))
    return out_l, out_r


def kernel(scores, bbox_deltas, im_info):
    return _run(scores, bbox_deltas, im_info)
